# baseline (speedup 1.0000x reference)
import jax
import jax.numpy as jnp
from jax.experimental import pallas as pl
from jax.experimental.pallas import tpu as pltpu

_TT = 2048
_K = 4


def _moe_kernel(x_ref, wr_ref, w1_ref, w2_ref, out_ref, scores_ref):
    e = pl.program_id(1)
    k = pl.program_id(2)

    @pl.when((e == 0) & (k == 0))
    def _():
        logits = jax.lax.dot_general(
            x_ref[...], wr_ref[...], (((1,), (1,)), ((), ())),
            preferred_element_type=jnp.float32)
        m = jnp.max(logits, axis=1, keepdims=True)
        p = jnp.exp(logits - m)
        scores_ref[...] = p / jnp.sum(p, axis=1, keepdims=True)
        out_ref[...] = jnp.zeros_like(out_ref)

    h = jax.lax.dot_general(
        x_ref[...], w1_ref[0], (((1,), (1,)), ((), ())),
        preferred_element_type=jnp.float32)
    h = h * jax.nn.sigmoid(h)
    o = jax.lax.dot_general(
        h, w2_ref[0], (((1,), (1,)), ((), ())),
        preferred_element_type=jnp.float32)
    sel = jax.lax.broadcasted_iota(jnp.int32, scores_ref.shape, 1) == e
    w = jnp.sum(jnp.where(sel, scores_ref[...], 0.0), axis=1, keepdims=True)
    out_ref[...] += w * o


def kernel(x, Wr, W1, W2):
    B, S, HID = x.shape
    E, INTER, _ = W1.shape
    T = B * S
    KC = INTER // _K

    out = pl.pallas_call(
        _moe_kernel,
        grid=(T // _TT, E, _K),
        in_specs=[
            pl.BlockSpec((_TT, HID), lambda i, e, k: (i, 0)),
            pl.BlockSpec((E, HID), lambda i, e, k: (0, 0)),
            pl.BlockSpec((1, KC, HID), lambda i, e, k: (e, k, 0)),
            pl.BlockSpec((1, HID, KC), lambda i, e, k: (e, 0, k)),
        ],
        out_specs=pl.BlockSpec((_TT, HID), lambda i, e, k: (i, 0)),
        out_shape=jax.ShapeDtypeStruct((T, HID), jnp.float32),
        scratch_shapes=[pltpu.VMEM((_TT, E), jnp.float32)],
        compiler_params=pltpu.CompilerParams(
            dimension_semantics=("arbitrary", "arbitrary", "arbitrary")),
    )(x.reshape(T, HID), Wr, W1, W2)
    return out.reshape(B, S, HID)


# speedup vs baseline: 1.0900x; 1.0900x over previous
import jax
import jax.numpy as jnp
from jax.experimental import pallas as pl
from jax.experimental.pallas import tpu as pltpu

_TT = 1024


def _moe_kernel(x_ref, wr_ref, w1_ref, w2_ref, out_ref, scores_ref):
    e = pl.program_id(1)

    @pl.when(e == 0)
    def _():
        logits = jax.lax.dot_general(
            x_ref[...], wr_ref[...], (((1,), (1,)), ((), ())),
            preferred_element_type=jnp.float32)
        m = jnp.max(logits, axis=1, keepdims=True)
        p = jnp.exp(logits - m)
        scores_ref[...] = p / jnp.sum(p, axis=1, keepdims=True)
        out_ref[...] = jnp.zeros_like(out_ref)

    h = jax.lax.dot_general(
        x_ref[...], w1_ref[0], (((1,), (1,)), ((), ())),
        preferred_element_type=jnp.float32)
    h = h * jax.nn.sigmoid(h)
    o = jax.lax.dot_general(
        h, w2_ref[0], (((1,), (1,)), ((), ())),
        preferred_element_type=jnp.float32)
    sel = jax.lax.broadcasted_iota(jnp.int32, scores_ref.shape, 1) == e
    w = jnp.sum(jnp.where(sel, scores_ref[...], 0.0), axis=1, keepdims=True)
    out_ref[...] += w * o


def kernel(x, Wr, W1, W2):
    B, S, HID = x.shape
    E, INTER, _ = W1.shape
    T = B * S

    out = pl.pallas_call(
        _moe_kernel,
        grid=(T // _TT, E),
        in_specs=[
            pl.BlockSpec((_TT, HID), lambda i, e: (i, 0)),
            pl.BlockSpec((E, HID), lambda i, e: (0, 0)),
            pl.BlockSpec((1, INTER, HID), lambda i, e: (e, 0, 0)),
            pl.BlockSpec((1, HID, INTER), lambda i, e: (e, 0, 0)),
        ],
        out_specs=pl.BlockSpec((_TT, HID), lambda i, e: (i, 0)),
        out_shape=jax.ShapeDtypeStruct((T, HID), jnp.float32),
        scratch_shapes=[pltpu.VMEM((_TT, E), jnp.float32)],
        compiler_params=pltpu.CompilerParams(
            dimension_semantics=("parallel", "arbitrary")),
    )(x.reshape(T, HID), Wr, W1, W2)
    return out.reshape(B, S, HID)
